# baseline (speedup 1.0000x reference)
import functools

import jax
import jax.numpy as jnp
from jax import lax
from jax.experimental import pallas as pl
from jax.experimental.pallas import tpu as pltpu
from jax.experimental.pallas import tpu_sc as plsc

DIM = 64
NUM_WORKERS = 32
CHUNK = 128


def _emb_body(n_chunks, idx_hbm, table_hbm, out_hbm, idx_v, rows_v, sem):
    cid = lax.axis_index("c")
    sid = lax.axis_index("s")
    wid = sid * 2 + cid
    pltpu.sync_copy(idx_hbm.at[pl.ds(wid * n_chunks, n_chunks)], idx_v)
    base = wid * n_chunks * CHUNK

    def step(j, carry):
        pltpu.async_copy(table_hbm.at[idx_v.at[j]], rows_v, sem).wait()
        pltpu.sync_copy(rows_v, out_hbm.at[pl.ds(base + j * CHUNK, CHUNK)])
        return carry

    lax.fori_loop(0, n_chunks, step, 0)


def kernel(token_ids, weight):
    b, s = token_ids.shape
    total = b * s
    assert total % (NUM_WORKERS * CHUNK) == 0
    n_chunks = total // (NUM_WORKERS * CHUNK)
    idx = token_ids.reshape(total // CHUNK, CHUNK).astype(jnp.int32)

    mesh = plsc.VectorSubcoreMesh(core_axis_name="c", subcore_axis_name="s")
    run = pl.kernel(
        functools.partial(_emb_body, n_chunks),
        out_type=jax.ShapeDtypeStruct((total, DIM), jnp.float32),
        mesh=mesh,
        scratch_types=[
            pltpu.VMEM((n_chunks, CHUNK), jnp.int32),
            pltpu.VMEM((CHUNK, DIM), jnp.float32),
            pltpu.SemaphoreType.DMA,
        ],
        compiler_params=pltpu.CompilerParams(use_tc_tiling_on_sc=False),
    )
    out = run(idx, weight)
    return out.reshape(b, s, DIM)


# speedup vs baseline: 1.1164x; 1.1164x over previous
import functools

import jax
import jax.numpy as jnp
from jax import lax
from jax.experimental import pallas as pl
from jax.experimental.pallas import tpu as pltpu
from jax.experimental.pallas import tpu_sc as plsc

DIM = 64
NUM_WORKERS = 32
CHUNK = 128
K = 4


def _emb_body(n_chunks, idx_hbm, table_hbm, out_hbm, idx_v, rows_v,
              sem_in, sem_out):
    cid = lax.axis_index("c")
    sid = lax.axis_index("s")
    wid = sid * 2 + cid
    pltpu.sync_copy(idx_hbm.at[pl.ds(wid * n_chunks, n_chunks)], idx_v)
    base = wid * n_chunks * CHUNK
    ng = n_chunks // K

    def fire_group(o, setoff):
        for b in range(K):
            pltpu.async_copy(
                table_hbm.at[idx_v.at[o * K + b]],
                rows_v.at[setoff * K + b], sem_in)

    def wait_one_gather():
        pltpu.make_async_copy(
            table_hbm.at[idx_v.at[0]], rows_v.at[0], sem_in).wait()

    def wait_one_store():
        pltpu.make_async_copy(
            rows_v.at[0], out_hbm.at[pl.ds(base, CHUNK)], sem_out).wait()

    fire_group(0, 0)

    def body(o, carry):
        cur = lax.rem(o, 2)

        @pl.when(o >= 1)
        def _():
            for _b in range(K):
                wait_one_store()

        @pl.when(o + 1 < ng)
        def _():
            fire_group(o + 1, 1 - cur)

        for b in range(K):
            j = o * K + b
            wait_one_gather()
            pltpu.async_copy(
                rows_v.at[cur * K + b],
                out_hbm.at[pl.ds(base + j * CHUNK, CHUNK)], sem_out)
        return carry

    lax.fori_loop(0, ng, body, 0)
    for _b in range(K):
        wait_one_store()


def kernel(token_ids, weight):
    b, s = token_ids.shape
    total = b * s
    assert total % (NUM_WORKERS * CHUNK) == 0
    n_chunks = total // (NUM_WORKERS * CHUNK)
    assert n_chunks % K == 0
    idx = token_ids.reshape(total // CHUNK, CHUNK).astype(jnp.int32)

    mesh = plsc.VectorSubcoreMesh(core_axis_name="c", subcore_axis_name="s")
    run = pl.kernel(
        functools.partial(_emb_body, n_chunks),
        out_type=jax.ShapeDtypeStruct((total, DIM), jnp.float32),
        mesh=mesh,
        scratch_types=[
            pltpu.VMEM((n_chunks, CHUNK), jnp.int32),
            pltpu.VMEM((2 * K, CHUNK, DIM), jnp.float32),
            pltpu.SemaphoreType.DMA,
            pltpu.SemaphoreType.DMA,
        ],
        compiler_params=pltpu.CompilerParams(use_tc_tiling_on_sc=False),
    )
    out = run(idx, weight)
    return out.reshape(b, s, DIM)
